# Initial kernel scaffold; baseline (speedup 1.0000x reference)
import jax
import jax.numpy as jnp
from jax.experimental import pallas as pl


def kernel(inputs, sa1, sa2, sa3, fc):
    raise NotImplementedError("write your pallas kernel here")


# speedup vs baseline: 4.1239x; 4.1239x over previous
import functools

import jax
import jax.numpy as jnp
from jax.experimental import pallas as pl

B = 8
N0 = 2048

def _relu(x):
    return jnp.maximum(x, 0.0)


def _dot(a, b):
    return jax.lax.dot_general(
        a, b, (((1,), (0,)), ((), ())), preferred_element_type=jnp.float32)


def _chunked_gather(tab, idx):
    n = tab.shape[1]
    acc = None
    for c in range(n // 128):
        part = jnp.take_along_axis(
            tab[:, c * 128:(c + 1) * 128],
            jnp.clip(idx - c * 128, 0, 127), axis=1)
        if acc is None:
            acc = part
        else:
            acc = jnp.where(idx >= c * 128, part, acc)
    return acc


def _fps_body(xyz_ref, ns_ref, *, npoint, n):
    X = xyz_ref[:, 0, :]
    Y = xyz_ref[:, 1, :]
    Z = xyz_ref[:, 2, :]
    lanes = jax.lax.broadcasted_iota(jnp.int32, (B, n), 1)

    def body(j, carry):
        dist, far = carry
        rows = []
        for _ in range(8):
            oh = lanes == far
            cx = jnp.sum(jnp.where(oh, X, 0.0), axis=1, keepdims=True)
            cy = jnp.sum(jnp.where(oh, Y, 0.0), axis=1, keepdims=True)
            cz = jnp.sum(jnp.where(oh, Z, 0.0), axis=1, keepdims=True)
            rows.append(jnp.concatenate([cx, cy, cz], axis=1)[:, None, :])
            d = (X - cx) ** 2 + (Y - cy) ** 2 + (Z - cz) ** 2
            dist = jnp.minimum(dist, d)
            m = jnp.max(dist, axis=1, keepdims=True)
            far = jnp.min(jnp.where(dist == m, lanes, n), axis=1,
                          keepdims=True)
        blk = jnp.concatenate(rows, axis=1)
        ns_ref[:, pl.ds(pl.multiple_of(j * 8, 8), 8), :] = blk
        return dist, far

    dist0 = jnp.full((B, n), 1e10, dtype=jnp.float32)
    far0 = jnp.zeros((B, 1), dtype=jnp.int32)
    jax.lax.fori_loop(0, npoint // 8, body, (dist0, far0))


def _fps(xyz_t, npoint):
    n = xyz_t.shape[2]
    nxyz_s = pl.pallas_call(
        functools.partial(_fps_body, npoint=npoint, n=n),
        out_shape=jax.ShapeDtypeStruct((B, npoint, 3), jnp.float32),
    )(xyz_t)
    return jnp.transpose(nxyz_s, (0, 2, 1)), nxyz_s


def _proj1_body(xyz_ref, nxyz_ref, wx_ref, a_ref, cs_ref):
    wx = wx_ref[...]
    a_ref[0] = _dot(wx, xyz_ref[0])
    cs_ref[0] = _dot(wx, nxyz_ref[0])


def _proj2_body(xyz_ref, pts_ref, nxyz_ref, wx_ref, wf_ref, a_ref, cs_ref):
    wx = wx_ref[...]
    a_ref[0] = _dot(wx, xyz_ref[0]) + _dot(wf_ref[...], pts_ref[0])
    cs_ref[0] = _dot(wx, nxyz_ref[0])


def _sa_body(xyz_ref, ncs_ref, a_ref, cs_ref, *refs, branches, n, sb):
    nb = len(branches)
    wts = refs[:3 * nb]
    out_ref = refs[3 * nb]

    xyz = xyz_ref[0]
    X = xyz[0:1, :]
    Y = xyz[1:2, :]
    Z = xyz[2:3, :]
    nc = ncs_ref[0, 0]
    cx = nc[:, 0:1]
    cy = nc[:, 1:2]
    cz = nc[:, 2:3]
    sqr = (cx - X) ** 2 + (cy - Y) ** 2 + (cz - Z) ** 2

    iota0 = jax.lax.broadcasted_iota(jnp.int32, (128, 128), 0)
    iota1 = jax.lax.broadcasted_iota(jnp.int32, (128, 128), 1)
    ut = (iota0 <= iota1).astype(jnp.float32)

    a = a_ref[0]
    cs = cs_ref[0, 0]

    c1_off = 0
    co_off = 0
    nsteps = n.bit_length()
    for bi, (r2, ns, c1, c2, c3) in enumerate(branches):
        w2t_ref, w3t_ref, bias_ref = wts[3 * bi:3 * bi + 3]
        mask = sqr <= jnp.float32(r2)
        mf = mask.astype(jnp.float32)
        off = jnp.zeros((sb, 1), jnp.float32)
        rank_chunks = []
        for c in range(n // 128):
            mc = mf[:, c * 128:(c + 1) * 128]
            rank_chunks.append(_dot(mc, ut) + off)
            off = off + jnp.sum(mc, axis=1, keepdims=True)
        rank = jnp.concatenate(rank_chunks, axis=1)
        count = off

        tgt = (jax.lax.broadcasted_iota(jnp.int32, (sb, ns), 1) + 1
               ).astype(jnp.float32)
        lo = jnp.zeros((sb, ns), jnp.int32)
        hi = jnp.full((sb, ns), n, jnp.int32)
        for _ in range(nsteps):
            act = lo < hi
            mid = (lo + hi) // 2
            rmid = _chunked_gather(rank, jnp.minimum(mid, n - 1))
            pred = act & (rmid < tgt)
            lo = jnp.where(pred, mid + 1, lo)
            hi = jnp.where(act & (~pred), mid, hi)
        valid = tgt <= count
        gidx = jnp.where(valid, lo, jnp.broadcast_to(lo[:, 0:1], (sb, ns)))
        gidx = jnp.minimum(gidx, n - 1)

        ab = a[c1_off:c1_off + c1, :]
        bias = bias_ref[...]
        b1 = bias[0:c1, :]
        b2 = bias[c1:c1 + c2, :]
        b3 = bias[c1 + c2:c1 + c2 + c3, :]
        cols = []
        for k in range(sb):
            idxk = jnp.broadcast_to(gidx[k:k + 1, :], (c1, ns))
            g = _chunked_gather(ab, idxk)
            ck = cs[c1_off:c1_off + c1, k:k + 1]
            cols.append(g - jnp.broadcast_to(ck, (c1, ns)))
        h1 = _relu(jnp.concatenate(cols, axis=1) + b1)
        h2 = _relu(_dot(w2t_ref[...], h1) + b2)
        h3 = _relu(_dot(w3t_ref[...], h2) + b3)
        for k in range(sb):
            m = jnp.max(h3[:, k * ns:(k + 1) * ns], axis=1)
            out_ref[0, 0, co_off:co_off + c3, k:k + 1] = m[:, None]
        c1_off += c1
        co_off += c3


def _sa_msg(xyz_t, nxyz_s, a, cs, branches, wts, s, sb):
    n = xyz_t.shape[2]
    ca = a.shape[1]
    co = sum(br[4] for br in branches)
    nsi = s // sb
    ncs4 = jnp.reshape(nxyz_s, (B, nsi, sb, 3))
    cs4 = jnp.transpose(jnp.reshape(cs, (B, ca, nsi, sb)), (0, 2, 1, 3))
    wt_specs = []
    for w in wts:
        wt_specs.append(pl.BlockSpec(w.shape, lambda b, si: (0, 0)))
    out4 = pl.pallas_call(
        functools.partial(_sa_body, branches=branches, n=n, sb=sb),
        grid=(B, nsi),
        in_specs=[
            pl.BlockSpec((1, 3, n), lambda b, si: (b, 0, 0)),
            pl.BlockSpec((1, 1, sb, 3), lambda b, si: (b, si, 0, 0)),
            pl.BlockSpec((1, ca, n), lambda b, si: (b, 0, 0)),
            pl.BlockSpec((1, 1, ca, sb), lambda b, si: (b, si, 0, 0)),
        ] + wt_specs,
        out_specs=pl.BlockSpec((1, 1, co, sb), lambda b, si: (b, si, 0, 0)),
        out_shape=jax.ShapeDtypeStruct((B, nsi, co, sb), jnp.float32),
    )(xyz_t, ncs4, a, cs4, *wts)
    return jnp.reshape(jnp.transpose(out4, (0, 2, 1, 3)), (B, co, s))


def _sa3_body(xyz_ref, pts_ref, w1_ref, w2_ref, w3_ref, bias_ref, out_ref):
    h = jnp.concatenate([xyz_ref[0], pts_ref[0]], axis=0)
    bias = bias_ref[...]
    b1 = bias[0:256, :]
    b2 = bias[256:256 + 512, :]
    b3 = bias[256 + 512:256 + 512 + 1024, :]
    h = _relu(_dot(w1_ref[...], h) + b1)
    h = _relu(_dot(w2_ref[...], h) + b2)
    h = _relu(_dot(w3_ref[...], h) + b3)
    out_ref[0] = jnp.max(h, axis=1)[:, None]


def _fc_body(p_ref, w1_ref, w2_ref, w3_ref, b1_ref, b2_ref, b3_ref, out_ref):
    f = p_ref[...]
    f = _relu(_dot(f, w1_ref[...]) + b1_ref[...])
    f = _relu(_dot(f, w2_ref[...]) + b2_ref[...])
    f = _relu(_dot(f, w3_ref[...]) + b3_ref[...])
    out_ref[...] = f


def _stack_wts(mlps):
    out = []
    for mlp in mlps:
        (w1, b1), (w2, b2), (w3, b3) = mlp
        bias = jnp.concatenate([b1, b2, b3])[:, None]
        out += [w2.T, w3.T, bias]
    return out


def kernel(inputs, sa1, sa2, sa3, fc):
    xyz0_t = jnp.transpose(inputs, (0, 2, 1))

    nxyz1_t, nxyz1_s = _fps(xyz0_t, 512)
    wx1 = jnp.concatenate([mlp[0][0].T for mlp in sa1], axis=0)
    a1, cs1 = pl.pallas_call(
        _proj1_body,
        grid=(B,),
        in_specs=[
            pl.BlockSpec((1, 3, N0), lambda b: (b, 0, 0)),
            pl.BlockSpec((1, 3, 512), lambda b: (b, 0, 0)),
            pl.BlockSpec(wx1.shape, lambda b: (0, 0)),
        ],
        out_specs=(
            pl.BlockSpec((1, 160, N0), lambda b: (b, 0, 0)),
            pl.BlockSpec((1, 160, 512), lambda b: (b, 0, 0)),
        ),
        out_shape=(
            jax.ShapeDtypeStruct((B, 160, N0), jnp.float32),
            jax.ShapeDtypeStruct((B, 160, 512), jnp.float32),
        ),
    )(xyz0_t, nxyz1_t, wx1)
    br1 = (
        (0.1 * 0.1, 16, 32, 32, 64),
        (0.2 * 0.2, 32, 64, 64, 128),
        (0.4 * 0.4, 128, 64, 96, 128),
    )
    pts1 = _sa_msg(xyz0_t, nxyz1_s, a1, cs1, br1, _stack_wts(sa1), 512, 8)

    nxyz2_t, nxyz2_s = _fps(nxyz1_t, 128)
    wx2 = jnp.concatenate([mlp[0][0][:3, :].T for mlp in sa2], axis=0)
    wf2 = jnp.concatenate([mlp[0][0][3:, :].T for mlp in sa2], axis=0)
    a2, cs2 = pl.pallas_call(
        _proj2_body,
        grid=(B,),
        in_specs=[
            pl.BlockSpec((1, 3, 512), lambda b: (b, 0, 0)),
            pl.BlockSpec((1, 320, 512), lambda b: (b, 0, 0)),
            pl.BlockSpec((1, 3, 128), lambda b: (b, 0, 0)),
            pl.BlockSpec(wx2.shape, lambda b: (0, 0)),
            pl.BlockSpec(wf2.shape, lambda b: (0, 0)),
        ],
        out_specs=(
            pl.BlockSpec((1, 320, 512), lambda b: (b, 0, 0)),
            pl.BlockSpec((1, 320, 128), lambda b: (b, 0, 0)),
        ),
        out_shape=(
            jax.ShapeDtypeStruct((B, 320, 512), jnp.float32),
            jax.ShapeDtypeStruct((B, 320, 128), jnp.float32),
        ),
    )(nxyz1_t, pts1, nxyz2_t, wx2, wf2)
    br2 = (
        (0.2 * 0.2, 32, 64, 64, 128),
        (0.4 * 0.4, 64, 128, 128, 256),
        (0.8 * 0.8, 128, 128, 128, 256),
    )
    pts2 = _sa_msg(nxyz1_t, nxyz2_s, a2, cs2, br2, _stack_wts(sa2), 128, 8)

    (w1, b1), (w2, b2), (w3, b3) = sa3
    bias3 = jnp.concatenate([b1, b2, b3])[:, None]
    pooled = pl.pallas_call(
        _sa3_body,
        grid=(B,),
        in_specs=[
            pl.BlockSpec((1, 3, 128), lambda b: (b, 0, 0)),
            pl.BlockSpec((1, 640, 128), lambda b: (b, 0, 0)),
            pl.BlockSpec((256, 643), lambda b: (0, 0)),
            pl.BlockSpec((512, 256), lambda b: (0, 0)),
            pl.BlockSpec((1024, 512), lambda b: (0, 0)),
            pl.BlockSpec((1792, 1), lambda b: (0, 0)),
        ],
        out_specs=pl.BlockSpec((1, 1024, 1), lambda b: (b, 0, 0)),
        out_shape=jax.ShapeDtypeStruct((B, 1024, 1), jnp.float32),
    )(nxyz2_t, pts2, w1.T, w2.T, w3.T, bias3)

    (fw1, fb1), (fw2, fb2), (fw3, fb3) = fc
    f = pl.pallas_call(
        _fc_body,
        out_shape=jax.ShapeDtypeStruct((B, 128), jnp.float32),
    )(pooled[:, :, 0], fw1, fw2, fw3, fb1[None, :], fb2[None, :], fb3[None, :])

    return (jnp.zeros((B, 1, 3), jnp.float32), f)


# speedup vs baseline: 7.0664x; 1.7136x over previous
import functools

import jax
import jax.numpy as jnp
from jax.experimental import pallas as pl

B = 8
N0 = 2048

def _relu(x):
    return jnp.maximum(x, 0.0)


def _dot(a, b):
    return jax.lax.dot_general(
        a, b, (((1,), (0,)), ((), ())), preferred_element_type=jnp.float32)


def _chunked_gather(tab, idx):
    n = tab.shape[1]
    acc = None
    for c in range(n // 128):
        part = jnp.take_along_axis(
            tab[:, c * 128:(c + 1) * 128],
            jnp.clip(idx - c * 128, 0, 127), axis=1)
        if acc is None:
            acc = part
        else:
            acc = jnp.where(idx >= c * 128, part, acc)
    return acc


def _fps_body(xyz_ref, ns_ref, *, npoint, n):
    X = xyz_ref[:, 0, :]
    Y = xyz_ref[:, 1, :]
    Z = xyz_ref[:, 2, :]
    lanes = jax.lax.broadcasted_iota(jnp.int32, (B, n), 1)

    def body(j, carry):
        dist, far = carry
        rows = []
        for _ in range(8):
            oh = lanes == far
            cx = jnp.sum(jnp.where(oh, X, 0.0), axis=1, keepdims=True)
            cy = jnp.sum(jnp.where(oh, Y, 0.0), axis=1, keepdims=True)
            cz = jnp.sum(jnp.where(oh, Z, 0.0), axis=1, keepdims=True)
            rows.append(jnp.concatenate([cx, cy, cz], axis=1)[:, None, :])
            d = (X - cx) ** 2 + (Y - cy) ** 2 + (Z - cz) ** 2
            dist = jnp.minimum(dist, d)
            m = jnp.max(dist, axis=1, keepdims=True)
            far = jnp.min(jnp.where(dist == m, lanes, n), axis=1,
                          keepdims=True)
        blk = jnp.concatenate(rows, axis=1)
        ns_ref[:, pl.ds(pl.multiple_of(j * 8, 8), 8), :] = blk
        return dist, far

    dist0 = jnp.full((B, n), 1e10, dtype=jnp.float32)
    far0 = jnp.zeros((B, 1), dtype=jnp.int32)
    jax.lax.fori_loop(0, npoint // 8, body, (dist0, far0))


def _fps(xyz_t, npoint):
    n = xyz_t.shape[2]
    nxyz_s = pl.pallas_call(
        functools.partial(_fps_body, npoint=npoint, n=n),
        out_shape=jax.ShapeDtypeStruct((B, npoint, 3), jnp.float32),
    )(xyz_t)
    return jnp.transpose(nxyz_s, (0, 2, 1)), nxyz_s


def _proj1_body(xyz_ref, nxyz_ref, wx_ref, a_ref, cs_ref):
    wx = wx_ref[...]
    a_ref[0] = _dot(wx, xyz_ref[0])
    cs_ref[0] = _dot(wx, nxyz_ref[0])


def _proj2_body(xyz_ref, pts_ref, nxyz_ref, wx_ref, wf_ref, a_ref, cs_ref):
    wx = wx_ref[...]
    a_ref[0] = _dot(wx, xyz_ref[0]) + _dot(wf_ref[...], pts_ref[0])
    cs_ref[0] = _dot(wx, nxyz_ref[0])


def _sa_body(xyz_ref, ncs_ref, *refs, branches, n, sb, raw):
    nb = len(branches)
    npb = 4 if raw else 3
    if raw:
        nct_ref = refs[0]
        wts = refs[1:1 + npb * nb]
        out_ref = refs[1 + npb * nb]
    else:
        a_ref, cs_ref = refs[0], refs[1]
        wts = refs[2:2 + npb * nb]
        out_ref = refs[2 + npb * nb]

    xyz = xyz_ref[0]
    X = xyz[0:1, :]
    Y = xyz[1:2, :]
    Z = xyz[2:3, :]
    nc = ncs_ref[0, 0]
    cx = nc[:, 0:1]
    cy = nc[:, 1:2]
    cz = nc[:, 2:3]
    sqr = (cx - X) ** 2 + (cy - Y) ** 2 + (cz - Z) ** 2

    iota0 = jax.lax.broadcasted_iota(jnp.int32, (128, 128), 0)
    iota1 = jax.lax.broadcasted_iota(jnp.int32, (128, 128), 1)
    ut = (iota0 <= iota1).astype(jnp.float32)
    nch = n // 128
    iota0c = jax.lax.broadcasted_iota(jnp.int32, (nch, nch), 0)
    iota1c = jax.lax.broadcasted_iota(jnp.int32, (nch, nch), 1)
    utx = (iota0c < iota1c).astype(jnp.float32)

    if raw:
        nct = nct_ref[0, 0]
        a = xyz
    else:
        a = a_ref[0]
        cs = cs_ref[0, 0]

    c1_off = 0
    co_off = 0
    nsteps = n.bit_length()
    for bi in range(nb):
        r2, ns, c1, c2, c3 = branches[bi]
        if raw:
            w1t_ref, w2t_ref, w3t_ref, bias_ref = wts[npb * bi:npb * bi + npb]
        else:
            w2t_ref, w3t_ref, bias_ref = wts[npb * bi:npb * bi + npb]
        mask = sqr <= jnp.float32(r2)
        mf = mask.astype(jnp.float32)
        pcs = [_dot(mf[:, c * 128:(c + 1) * 128], ut) for c in range(nch)]
        sums = jnp.concatenate([p[:, -1:] for p in pcs], axis=1)
        offs = _dot(sums, utx)
        rank = jnp.concatenate(
            [pcs[c] + offs[:, c:c + 1] for c in range(nch)], axis=1)
        count = offs[:, -1:] + sums[:, -1:]

        tgt = (jax.lax.broadcasted_iota(jnp.int32, (sb, ns), 1) + 1
               ).astype(jnp.float32)
        lo = jnp.zeros((sb, ns), jnp.int32)
        hi = jnp.full((sb, ns), n, jnp.int32)
        for _ in range(nsteps):
            act = lo < hi
            mid = (lo + hi) // 2
            rmid = _chunked_gather(rank, jnp.minimum(mid, n - 1))
            pred = act & (rmid < tgt)
            lo = jnp.where(pred, mid + 1, lo)
            hi = jnp.where(act & (~pred), mid, hi)
        valid = tgt <= count
        gidx = jnp.where(valid, lo, jnp.broadcast_to(lo[:, 0:1], (sb, ns)))
        gidx = jnp.minimum(gidx, n - 1)

        bias = bias_ref[...]
        b1 = bias[0:c1, :]
        b2 = bias[c1:c1 + c2, :]
        b3 = bias[c1 + c2:c1 + c2 + c3, :]
        if raw:
            cols = []
            for k in range(sb):
                idxk = jnp.broadcast_to(gidx[k:k + 1, :], (3, ns))
                g = _chunked_gather(xyz, idxk)
                ck = nct[:, k:k + 1]
                cols.append(g - jnp.broadcast_to(ck, (3, ns)))
            pre = jnp.concatenate(cols, axis=1)
            h1 = _relu(_dot(w1t_ref[...], pre) + b1)
        else:
            ab = a[c1_off:c1_off + c1, :]
            cols = []
            for k in range(sb):
                idxk = jnp.broadcast_to(gidx[k:k + 1, :], (c1, ns))
                g = _chunked_gather(ab, idxk)
                ck = cs[c1_off:c1_off + c1, k:k + 1]
                cols.append(g - jnp.broadcast_to(ck, (c1, ns)))
            h1 = _relu(jnp.concatenate(cols, axis=1) + b1)
        h2 = _relu(_dot(w2t_ref[...], h1) + b2)
        h3 = _relu(_dot(w3t_ref[...], h2) + b3)
        for k in range(sb):
            m = jnp.max(h3[:, k * ns:(k + 1) * ns], axis=1)
            out_ref[0, 0, co_off:co_off + c3, k:k + 1] = m[:, None]
        c1_off += c1
        co_off += c3


def _sa_msg(xyz_t, nxyz_t, nxyz_s, a, cs, branches, wts, s, sb):
    n = xyz_t.shape[2]
    raw = a is None
    co = sum(br[4] for br in branches)
    nsi = s // sb
    ncs4 = jnp.reshape(nxyz_s, (B, nsi, sb, 3))
    wt_specs = [pl.BlockSpec(w.shape, lambda b, si: (0, 0)) for w in wts]
    if raw:
        nct4 = jnp.transpose(
            jnp.reshape(nxyz_t, (B, 3, nsi, sb)), (0, 2, 1, 3))
        extra = [nct4]
        extra_specs = [pl.BlockSpec((1, 1, 3, sb), lambda b, si: (b, si, 0, 0))]
    else:
        ca = a.shape[1]
        cs4 = jnp.transpose(jnp.reshape(cs, (B, ca, nsi, sb)), (0, 2, 1, 3))
        extra = [a, cs4]
        extra_specs = [
            pl.BlockSpec((1, ca, n), lambda b, si: (b, 0, 0)),
            pl.BlockSpec((1, 1, ca, sb), lambda b, si: (b, si, 0, 0)),
        ]
    out4 = pl.pallas_call(
        functools.partial(_sa_body, branches=branches, n=n, sb=sb, raw=raw),
        grid=(B, nsi),
        in_specs=[
            pl.BlockSpec((1, 3, n), lambda b, si: (b, 0, 0)),
            pl.BlockSpec((1, 1, sb, 3), lambda b, si: (b, si, 0, 0)),
        ] + extra_specs + wt_specs,
        out_specs=pl.BlockSpec((1, 1, co, sb), lambda b, si: (b, si, 0, 0)),
        out_shape=jax.ShapeDtypeStruct((B, nsi, co, sb), jnp.float32),
    )(xyz_t, ncs4, *extra, *wts)
    return jnp.reshape(jnp.transpose(out4, (0, 2, 1, 3)), (B, co, s))


def _sa3_body(xyz_ref, pts_ref, w1_ref, w2_ref, w3_ref, bias_ref, out_ref):
    h = jnp.concatenate([xyz_ref[0], pts_ref[0]], axis=0)
    bias = bias_ref[...]
    b1 = bias[0:256, :]
    b2 = bias[256:256 + 512, :]
    b3 = bias[256 + 512:256 + 512 + 1024, :]
    h = _relu(_dot(w1_ref[...], h) + b1)
    h = _relu(_dot(w2_ref[...], h) + b2)
    h = _relu(_dot(w3_ref[...], h) + b3)
    out_ref[0] = jnp.max(h, axis=1)[:, None]


def _fc_body(p_ref, w1_ref, w2_ref, w3_ref, b1_ref, b2_ref, b3_ref, out_ref):
    f = p_ref[...]
    f = _relu(_dot(f, w1_ref[...]) + b1_ref[...])
    f = _relu(_dot(f, w2_ref[...]) + b2_ref[...])
    f = _relu(_dot(f, w3_ref[...]) + b3_ref[...])
    out_ref[...] = f


def _stack_wts(mlps, with_w1=False):
    out = []
    for mlp in mlps:
        (w1, b1), (w2, b2), (w3, b3) = mlp
        bias = jnp.concatenate([b1, b2, b3])[:, None]
        if with_w1:
            out += [w1.T, w2.T, w3.T, bias]
        else:
            out += [w2.T, w3.T, bias]
    return out


def kernel(inputs, sa1, sa2, sa3, fc):
    xyz0_t = jnp.transpose(inputs, (0, 2, 1))

    nxyz1_t, nxyz1_s = _fps(xyz0_t, 512)
    br1 = (
        (0.1 * 0.1, 16, 32, 32, 64),
        (0.2 * 0.2, 32, 64, 64, 128),
        (0.4 * 0.4, 128, 64, 96, 128),
    )
    pts1 = _sa_msg(xyz0_t, nxyz1_t, nxyz1_s, None, None, br1,
                   _stack_wts(sa1, with_w1=True), 512, 16)

    nxyz2_t, nxyz2_s = _fps(nxyz1_t, 128)
    wx2 = jnp.concatenate([mlp[0][0][:3, :].T for mlp in sa2], axis=0)
    wf2 = jnp.concatenate([mlp[0][0][3:, :].T for mlp in sa2], axis=0)
    a2, cs2 = pl.pallas_call(
        _proj2_body,
        grid=(B,),
        in_specs=[
            pl.BlockSpec((1, 3, 512), lambda b: (b, 0, 0)),
            pl.BlockSpec((1, 320, 512), lambda b: (b, 0, 0)),
            pl.BlockSpec((1, 3, 128), lambda b: (b, 0, 0)),
            pl.BlockSpec(wx2.shape, lambda b: (0, 0)),
            pl.BlockSpec(wf2.shape, lambda b: (0, 0)),
        ],
        out_specs=(
            pl.BlockSpec((1, 320, 512), lambda b: (b, 0, 0)),
            pl.BlockSpec((1, 320, 128), lambda b: (b, 0, 0)),
        ),
        out_shape=(
            jax.ShapeDtypeStruct((B, 320, 512), jnp.float32),
            jax.ShapeDtypeStruct((B, 320, 128), jnp.float32),
        ),
    )(nxyz1_t, pts1, nxyz2_t, wx2, wf2)
    br2 = (
        (0.2 * 0.2, 32, 64, 64, 128),
        (0.4 * 0.4, 64, 128, 128, 256),
        (0.8 * 0.8, 128, 128, 128, 256),
    )
    pts2 = _sa_msg(nxyz1_t, nxyz2_t, nxyz2_s, a2, cs2, br2,
                   _stack_wts(sa2), 128, 16)

    (w1, b1), (w2, b2), (w3, b3) = sa3
    bias3 = jnp.concatenate([b1, b2, b3])[:, None]
    pooled = pl.pallas_call(
        _sa3_body,
        grid=(B,),
        in_specs=[
            pl.BlockSpec((1, 3, 128), lambda b: (b, 0, 0)),
            pl.BlockSpec((1, 640, 128), lambda b: (b, 0, 0)),
            pl.BlockSpec((256, 643), lambda b: (0, 0)),
            pl.BlockSpec((512, 256), lambda b: (0, 0)),
            pl.BlockSpec((1024, 512), lambda b: (0, 0)),
            pl.BlockSpec((1792, 1), lambda b: (0, 0)),
        ],
        out_specs=pl.BlockSpec((1, 1024, 1), lambda b: (b, 0, 0)),
        out_shape=jax.ShapeDtypeStruct((B, 1024, 1), jnp.float32),
    )(nxyz2_t, pts2, w1.T, w2.T, w3.T, bias3)

    (fw1, fb1), (fw2, fb2), (fw3, fb3) = fc
    f = pl.pallas_call(
        _fc_body,
        out_shape=jax.ShapeDtypeStruct((B, 128), jnp.float32),
    )(pooled[:, :, 0], fw1, fw2, fw3, fb1[None, :], fb2[None, :], fb3[None, :])

    return (jnp.zeros((B, 1, 3), jnp.float32), f)


# speedup vs baseline: 9.4160x; 1.3325x over previous
import functools

import jax
import jax.numpy as jnp
from jax.experimental import pallas as pl

B = 8
N0 = 2048

def _relu(x):
    return jnp.maximum(x, 0.0)


def _dot(a, b):
    return jax.lax.dot_general(
        a, b, (((1,), (0,)), ((), ())), preferred_element_type=jnp.float32)


def _chunked_gather(tab, idx):
    n = tab.shape[1]
    acc = None
    for c in range(n // 128):
        part = jnp.take_along_axis(
            tab[:, c * 128:(c + 1) * 128],
            jnp.clip(idx - c * 128, 0, 127), axis=1)
        if acc is None:
            acc = part
        else:
            acc = jnp.where(idx >= c * 128, part, acc)
    return acc


def _fps_body(xyz_ref, ns_ref, *, npoint, n):
    X = xyz_ref[:, 0, :]
    Y = xyz_ref[:, 1, :]
    Z = xyz_ref[:, 2, :]
    lanes = jax.lax.broadcasted_iota(jnp.int32, (B, n), 1)

    def body(j, carry):
        dist, far = carry
        rows = []
        for _ in range(8):
            oh = lanes == far
            cx = jnp.sum(jnp.where(oh, X, 0.0), axis=1, keepdims=True)
            cy = jnp.sum(jnp.where(oh, Y, 0.0), axis=1, keepdims=True)
            cz = jnp.sum(jnp.where(oh, Z, 0.0), axis=1, keepdims=True)
            rows.append(jnp.concatenate([cx, cy, cz], axis=1)[:, None, :])
            d = (X - cx) ** 2 + (Y - cy) ** 2 + (Z - cz) ** 2
            dist = jnp.minimum(dist, d)
            m = jnp.max(dist, axis=1, keepdims=True)
            far = jnp.min(jnp.where(dist == m, lanes, n), axis=1,
                          keepdims=True)
        blk = jnp.concatenate(rows, axis=1)
        ns_ref[:, pl.ds(pl.multiple_of(j * 8, 8), 8), :] = blk
        return dist, far

    dist0 = jnp.full((B, n), 1e10, dtype=jnp.float32)
    far0 = jnp.zeros((B, 1), dtype=jnp.int32)
    jax.lax.fori_loop(0, npoint // 8, body, (dist0, far0))


def _fps(xyz_t, npoint):
    n = xyz_t.shape[2]
    nxyz_s = pl.pallas_call(
        functools.partial(_fps_body, npoint=npoint, n=n),
        out_shape=jax.ShapeDtypeStruct((B, npoint, 3), jnp.float32),
    )(xyz_t)
    return jnp.transpose(nxyz_s, (0, 2, 1)), nxyz_s


def _proj1_body(xyz_ref, nxyz_ref, wx_ref, a_ref, cs_ref):
    wx = wx_ref[...]
    a_ref[0] = _dot(wx, xyz_ref[0])
    cs_ref[0] = _dot(wx, nxyz_ref[0])


def _proj2_body(xyz_ref, pts_ref, nxyz_ref, wx_ref, wf_ref, a_ref, cs_ref):
    wx = wx_ref[...]
    a_ref[0] = _dot(wx, xyz_ref[0]) + _dot(wf_ref[...], pts_ref[0])
    cs_ref[0] = _dot(wx, nxyz_ref[0])


def _sa_body(xyz_ref, ncs_ref, *refs, branches, n, sb, raw):
    nb = len(branches)
    npb = 4 if raw else 3
    if raw:
        nct_ref = refs[0]
        wts = refs[1:1 + npb * nb]
        out_ref = refs[1 + npb * nb]
    else:
        a_ref, cs_ref = refs[0], refs[1]
        wts = refs[2:2 + npb * nb]
        out_ref = refs[2 + npb * nb]

    xyz = xyz_ref[0]
    X = xyz[0:1, :]
    Y = xyz[1:2, :]
    Z = xyz[2:3, :]
    nc = ncs_ref[0, 0]
    cx = nc[:, 0:1]
    cy = nc[:, 1:2]
    cz = nc[:, 2:3]
    sqr = (cx - X) ** 2 + (cy - Y) ** 2 + (cz - Z) ** 2

    iota0 = jax.lax.broadcasted_iota(jnp.int32, (128, 128), 0)
    iota1 = jax.lax.broadcasted_iota(jnp.int32, (128, 128), 1)
    ut = (iota0 <= iota1).astype(jnp.float32)
    nch = n // 128
    iota0c = jax.lax.broadcasted_iota(jnp.int32, (nch, nch), 0)
    iota1c = jax.lax.broadcasted_iota(jnp.int32, (nch, nch), 1)
    utx = (iota0c < iota1c).astype(jnp.float32)

    if raw:
        nct = nct_ref[0, 0]
        a = xyz
    else:
        a = a_ref[0]
        cs = cs_ref[0, 0]

    c1_off = 0
    co_off = 0
    nsteps = n.bit_length()
    for bi in range(nb):
        r2, ns, c1, c2, c3 = branches[bi]
        if raw:
            w1t_ref, w2t_ref, w3t_ref, bias_ref = wts[npb * bi:npb * bi + npb]
        else:
            w2t_ref, w3t_ref, bias_ref = wts[npb * bi:npb * bi + npb]
        mask = sqr <= jnp.float32(r2)
        mf = mask.astype(jnp.float32)
        pcs = [_dot(mf[:, c * 128:(c + 1) * 128], ut) for c in range(nch)]
        sums = jnp.concatenate([p[:, -1:] for p in pcs], axis=1)
        offs = _dot(sums, utx)
        rank = jnp.concatenate(
            [pcs[c] + offs[:, c:c + 1] for c in range(nch)], axis=1)
        count = offs[:, -1:] + sums[:, -1:]

        tgt = (jax.lax.broadcasted_iota(jnp.int32, (sb, ns), 1) + 1
               ).astype(jnp.float32)
        lo = jnp.zeros((sb, ns), jnp.int32)
        hi = jnp.full((sb, ns), n, jnp.int32)
        for _ in range(nsteps):
            act = lo < hi
            mid = (lo + hi) // 2
            rmid = _chunked_gather(rank, jnp.minimum(mid, n - 1))
            pred = act & (rmid < tgt)
            lo = jnp.where(pred, mid + 1, lo)
            hi = jnp.where(act & (~pred), mid, hi)
        valid = tgt <= count
        gidx = jnp.where(valid, lo, jnp.broadcast_to(lo[:, 0:1], (sb, ns)))
        gidx = jnp.minimum(gidx, n - 1)

        bias = bias_ref[...]
        b1 = bias[0:c1, :]
        b2 = bias[c1:c1 + c2, :]
        b3 = bias[c1 + c2:c1 + c2 + c3, :]
        if raw:
            cols = []
            for k in range(sb):
                idxk = jnp.broadcast_to(gidx[k:k + 1, :], (3, ns))
                g = _chunked_gather(xyz, idxk)
                ck = nct[:, k:k + 1]
                cols.append(g - jnp.broadcast_to(ck, (3, ns)))
            pre = jnp.concatenate(cols, axis=1)
            h1 = _relu(_dot(w1t_ref[...], pre) + b1)
        else:
            ab = a[c1_off:c1_off + c1, :]
            cols = []
            for k in range(sb):
                idxk = jnp.broadcast_to(gidx[k:k + 1, :], (c1, ns))
                g = _chunked_gather(ab, idxk)
                ck = cs[c1_off:c1_off + c1, k:k + 1]
                cols.append(g - jnp.broadcast_to(ck, (c1, ns)))
            h1 = _relu(jnp.concatenate(cols, axis=1) + b1)
        h2 = _relu(_dot(w2t_ref[...], h1) + b2)
        h3 = _relu(_dot(w3t_ref[...], h2) + b3)
        for k in range(sb):
            m = jnp.max(h3[:, k * ns:(k + 1) * ns], axis=1)
            out_ref[0, 0, co_off:co_off + c3, k:k + 1] = m[:, None]
        c1_off += c1
        co_off += c3


def _sa_msg(xyz_t, nxyz_t, nxyz_s, a, cs, branches, wts, s, sb):
    n = xyz_t.shape[2]
    raw = a is None
    co = sum(br[4] for br in branches)
    nsi = s // sb
    ncs4 = jnp.reshape(nxyz_s, (B, nsi, sb, 3))
    wt_specs = [pl.BlockSpec(w.shape, lambda b, si: (0, 0)) for w in wts]
    if raw:
        nct4 = jnp.transpose(
            jnp.reshape(nxyz_t, (B, 3, nsi, sb)), (0, 2, 1, 3))
        extra = [nct4]
        extra_specs = [pl.BlockSpec((1, 1, 3, sb), lambda b, si: (b, si, 0, 0))]
    else:
        ca = a.shape[1]
        cs4 = jnp.transpose(jnp.reshape(cs, (B, ca, nsi, sb)), (0, 2, 1, 3))
        extra = [a, cs4]
        extra_specs = [
            pl.BlockSpec((1, ca, n), lambda b, si: (b, 0, 0)),
            pl.BlockSpec((1, 1, ca, sb), lambda b, si: (b, si, 0, 0)),
        ]
    out4 = pl.pallas_call(
        functools.partial(_sa_body, branches=branches, n=n, sb=sb, raw=raw),
        grid=(B, nsi),
        in_specs=[
            pl.BlockSpec((1, 3, n), lambda b, si: (b, 0, 0)),
            pl.BlockSpec((1, 1, sb, 3), lambda b, si: (b, si, 0, 0)),
        ] + extra_specs + wt_specs,
        out_specs=pl.BlockSpec((1, 1, co, sb), lambda b, si: (b, si, 0, 0)),
        out_shape=jax.ShapeDtypeStruct((B, nsi, co, sb), jnp.float32),
    )(xyz_t, ncs4, *extra, *wts)
    return jnp.reshape(jnp.transpose(out4, (0, 2, 1, 3)), (B, co, s))


def _sa3_body(xyz_ref, pts_ref, w1_ref, w2_ref, w3_ref, bias_ref, out_ref):
    h = jnp.concatenate([xyz_ref[0], pts_ref[0]], axis=0)
    bias = bias_ref[...]
    b1 = bias[0:256, :]
    b2 = bias[256:256 + 512, :]
    b3 = bias[256 + 512:256 + 512 + 1024, :]
    h = _relu(_dot(w1_ref[...], h) + b1)
    h = _relu(_dot(w2_ref[...], h) + b2)
    h = _relu(_dot(w3_ref[...], h) + b3)
    out_ref[0] = jnp.max(h, axis=1)[:, None]


def _fc_body(p_ref, w1_ref, w2_ref, w3_ref, b1_ref, b2_ref, b3_ref, out_ref):
    f = p_ref[...]
    f = _relu(_dot(f, w1_ref[...]) + b1_ref[...])
    f = _relu(_dot(f, w2_ref[...]) + b2_ref[...])
    f = _relu(_dot(f, w3_ref[...]) + b3_ref[...])
    out_ref[...] = f


def _stack_wts(mlps, with_w1=False):
    out = []
    for mlp in mlps:
        (w1, b1), (w2, b2), (w3, b3) = mlp
        bias = jnp.concatenate([b1, b2, b3])[:, None]
        if with_w1:
            out += [w1.T, w2.T, w3.T, bias]
        else:
            out += [w2.T, w3.T, bias]
    return out


def kernel(inputs, sa1, sa2, sa3, fc):
    xyz0_t = jnp.transpose(inputs, (0, 2, 1))

    nxyz1_t, nxyz1_s = _fps(xyz0_t, 512)
    br1 = (
        (0.1 * 0.1, 16, 32, 32, 64),
        (0.2 * 0.2, 32, 64, 64, 128),
        (0.4 * 0.4, 128, 64, 96, 128),
    )
    pts1 = _sa_msg(xyz0_t, nxyz1_t, nxyz1_s, None, None, br1,
                   _stack_wts(sa1, with_w1=True), 512, 32)

    nxyz2_t, nxyz2_s = _fps(nxyz1_t, 128)
    wx2 = jnp.concatenate([mlp[0][0][:3, :].T for mlp in sa2], axis=0)
    wf2 = jnp.concatenate([mlp[0][0][3:, :].T for mlp in sa2], axis=0)
    a2, cs2 = pl.pallas_call(
        _proj2_body,
        grid=(B,),
        in_specs=[
            pl.BlockSpec((1, 3, 512), lambda b: (b, 0, 0)),
            pl.BlockSpec((1, 320, 512), lambda b: (b, 0, 0)),
            pl.BlockSpec((1, 3, 128), lambda b: (b, 0, 0)),
            pl.BlockSpec(wx2.shape, lambda b: (0, 0)),
            pl.BlockSpec(wf2.shape, lambda b: (0, 0)),
        ],
        out_specs=(
            pl.BlockSpec((1, 320, 512), lambda b: (b, 0, 0)),
            pl.BlockSpec((1, 320, 128), lambda b: (b, 0, 0)),
        ),
        out_shape=(
            jax.ShapeDtypeStruct((B, 320, 512), jnp.float32),
            jax.ShapeDtypeStruct((B, 320, 128), jnp.float32),
        ),
    )(nxyz1_t, pts1, nxyz2_t, wx2, wf2)
    br2 = (
        (0.2 * 0.2, 32, 64, 64, 128),
        (0.4 * 0.4, 64, 128, 128, 256),
        (0.8 * 0.8, 128, 128, 128, 256),
    )
    pts2 = _sa_msg(nxyz1_t, nxyz2_t, nxyz2_s, a2, cs2, br2,
                   _stack_wts(sa2), 128, 32)

    (w1, b1), (w2, b2), (w3, b3) = sa3
    bias3 = jnp.concatenate([b1, b2, b3])[:, None]
    pooled = pl.pallas_call(
        _sa3_body,
        grid=(B,),
        in_specs=[
            pl.BlockSpec((1, 3, 128), lambda b: (b, 0, 0)),
            pl.BlockSpec((1, 640, 128), lambda b: (b, 0, 0)),
            pl.BlockSpec((256, 643), lambda b: (0, 0)),
            pl.BlockSpec((512, 256), lambda b: (0, 0)),
            pl.BlockSpec((1024, 512), lambda b: (0, 0)),
            pl.BlockSpec((1792, 1), lambda b: (0, 0)),
        ],
        out_specs=pl.BlockSpec((1, 1024, 1), lambda b: (b, 0, 0)),
        out_shape=jax.ShapeDtypeStruct((B, 1024, 1), jnp.float32),
    )(nxyz2_t, pts2, w1.T, w2.T, w3.T, bias3)

    (fw1, fb1), (fw2, fb2), (fw3, fb3) = fc
    f = pl.pallas_call(
        _fc_body,
        out_shape=jax.ShapeDtypeStruct((B, 128), jnp.float32),
    )(pooled[:, :, 0], fw1, fw2, fw3, fb1[None, :], fb2[None, :], fb3[None, :])

    return (jnp.zeros((B, 1, 3), jnp.float32), f)


# speedup vs baseline: 10.3058x; 1.0945x over previous
import functools

import jax
import jax.numpy as jnp
from jax.experimental import pallas as pl

B = 8
N0 = 2048

def _relu(x):
    return jnp.maximum(x, 0.0)


def _dot(a, b):
    return jax.lax.dot_general(
        a, b, (((1,), (0,)), ((), ())), preferred_element_type=jnp.float32)


def _chunked_gather(tab, idx):
    n = tab.shape[1]
    acc = None
    for c in range(n // 128):
        part = jnp.take_along_axis(
            tab[:, c * 128:(c + 1) * 128],
            jnp.clip(idx - c * 128, 0, 127), axis=1)
        if acc is None:
            acc = part
        else:
            acc = jnp.where(idx >= c * 128, part, acc)
    return acc


def _fps_body(xyz_ref, ns_ref, *, npoint, n):
    X = xyz_ref[:, 0, :]
    Y = xyz_ref[:, 1, :]
    Z = xyz_ref[:, 2, :]
    lanes = jax.lax.broadcasted_iota(jnp.int32, (B, n), 1)

    def body(j, carry):
        dist, far = carry
        rows = []
        for _ in range(8):
            oh = lanes == far
            cx = jnp.sum(jnp.where(oh, X, 0.0), axis=1, keepdims=True)
            cy = jnp.sum(jnp.where(oh, Y, 0.0), axis=1, keepdims=True)
            cz = jnp.sum(jnp.where(oh, Z, 0.0), axis=1, keepdims=True)
            rows.append(jnp.concatenate([cx, cy, cz], axis=1)[:, None, :])
            d = (X - cx) ** 2 + (Y - cy) ** 2 + (Z - cz) ** 2
            dist = jnp.minimum(dist, d)
            m = jnp.max(dist, axis=1, keepdims=True)
            far = jnp.min(jnp.where(dist == m, lanes, n), axis=1,
                          keepdims=True)
        blk = jnp.concatenate(rows, axis=1)
        ns_ref[:, pl.ds(pl.multiple_of(j * 8, 8), 8), :] = blk
        return dist, far

    dist0 = jnp.full((B, n), 1e10, dtype=jnp.float32)
    far0 = jnp.zeros((B, 1), dtype=jnp.int32)
    jax.lax.fori_loop(0, npoint // 8, body, (dist0, far0))


def _fps(xyz_t, npoint):
    n = xyz_t.shape[2]
    nxyz_s = pl.pallas_call(
        functools.partial(_fps_body, npoint=npoint, n=n),
        out_shape=jax.ShapeDtypeStruct((B, npoint, 3), jnp.float32),
    )(xyz_t)
    return jnp.transpose(nxyz_s, (0, 2, 1)), nxyz_s


def _proj1_body(xyz_ref, nxyz_ref, wx_ref, a_ref, cs_ref):
    wx = wx_ref[...]
    a_ref[0] = _dot(wx, xyz_ref[0])
    cs_ref[0] = _dot(wx, nxyz_ref[0])


def _proj2_body(xyz_ref, pts_ref, nxyz_ref, wx_ref, wf_ref, a_ref, cs_ref):
    wx = wx_ref[...]
    a_ref[0] = _dot(wx, xyz_ref[0]) + _dot(wf_ref[...], pts_ref[0])
    cs_ref[0] = _dot(wx, nxyz_ref[0])


def _sa_body(xyz_ref, ncs_ref, *refs, branches, n, sb, raw):
    nb = len(branches)
    npb = 4 if raw else 3
    if raw:
        nct_ref = refs[0]
        wts = refs[1:1 + npb * nb]
        out_ref = refs[1 + npb * nb]
    else:
        a_ref, cs_ref = refs[0], refs[1]
        wts = refs[2:2 + npb * nb]
        out_ref = refs[2 + npb * nb]

    xyz = xyz_ref[0]
    X = xyz[0:1, :]
    Y = xyz[1:2, :]
    Z = xyz[2:3, :]
    nc = ncs_ref[0, 0]
    cx = nc[:, 0:1]
    cy = nc[:, 1:2]
    cz = nc[:, 2:3]
    sqr = (cx - X) ** 2 + (cy - Y) ** 2 + (cz - Z) ** 2

    iota0 = jax.lax.broadcasted_iota(jnp.int32, (128, 128), 0)
    iota1 = jax.lax.broadcasted_iota(jnp.int32, (128, 128), 1)
    ut = (iota0 <= iota1).astype(jnp.float32)
    nch = n // 128
    iota0c = jax.lax.broadcasted_iota(jnp.int32, (nch, nch), 0)
    iota1c = jax.lax.broadcasted_iota(jnp.int32, (nch, nch), 1)
    utx = (iota0c < iota1c).astype(jnp.float32)

    if raw:
        nct = nct_ref[0, 0]
        a = xyz
    else:
        a = a_ref[0]
        cs = cs_ref[0, 0]

    c1_off = 0
    co_off = 0
    nsteps = 7
    for bi in range(nb):
        r2, ns, c1, c2, c3 = branches[bi]
        if raw:
            w1t_ref, w2t_ref, w3t_ref, bias_ref = wts[npb * bi:npb * bi + npb]
        else:
            w2t_ref, w3t_ref, bias_ref = wts[npb * bi:npb * bi + npb]
        mask = sqr <= jnp.float32(r2)
        mf = mask.astype(jnp.float32)
        pcs = [_dot(mf[:, c * 128:(c + 1) * 128], ut) for c in range(nch)]
        sums = jnp.concatenate([p[:, -1:] for p in pcs], axis=1)
        offs = _dot(sums, utx)
        rank = jnp.concatenate(
            [pcs[c] + offs[:, c:c + 1] for c in range(nch)], axis=1)
        count = offs[:, -1:] + sums[:, -1:]

        tgt = (jax.lax.broadcasted_iota(jnp.int32, (sb, ns), 1) + 1
               ).astype(jnp.float32)
        ccum = offs + sums
        cid = jnp.zeros((sb, ns), jnp.int32)
        for c in range(nch):
            cid = cid + (ccum[:, c:c + 1] < tgt).astype(jnp.int32)
        lo = cid * 128
        hi = jnp.minimum(lo + 127, n)
        for _ in range(nsteps):
            act = lo < hi
            mid = (lo + hi) // 2
            rmid = _chunked_gather(rank, jnp.minimum(mid, n - 1))
            pred = act & (rmid < tgt)
            lo = jnp.where(pred, mid + 1, lo)
            hi = jnp.where(act & (~pred), mid, hi)
        valid = tgt <= count
        gidx = jnp.where(valid, lo, jnp.broadcast_to(lo[:, 0:1], (sb, ns)))
        gidx = jnp.minimum(gidx, n - 1)

        bias = bias_ref[...]
        b1 = bias[0:c1, :]
        b2 = bias[c1:c1 + c2, :]
        b3 = bias[c1 + c2:c1 + c2 + c3, :]
        if raw:
            cols = []
            for k in range(sb):
                idxk = jnp.broadcast_to(gidx[k:k + 1, :], (3, ns))
                g = _chunked_gather(xyz, idxk)
                ck = nct[:, k:k + 1]
                cols.append(g - jnp.broadcast_to(ck, (3, ns)))
            pre = jnp.concatenate(cols, axis=1)
            h1 = _relu(_dot(w1t_ref[...], pre) + b1)
        else:
            ab = a[c1_off:c1_off + c1, :]
            cols = []
            for k in range(sb):
                idxk = jnp.broadcast_to(gidx[k:k + 1, :], (c1, ns))
                g = _chunked_gather(ab, idxk)
                ck = cs[c1_off:c1_off + c1, k:k + 1]
                cols.append(g - jnp.broadcast_to(ck, (c1, ns)))
            h1 = _relu(jnp.concatenate(cols, axis=1) + b1)
        h2 = _relu(_dot(w2t_ref[...], h1) + b2)
        h3 = _relu(_dot(w3t_ref[...], h2) + b3)
        for k in range(sb):
            m = jnp.max(h3[:, k * ns:(k + 1) * ns], axis=1)
            out_ref[0, 0, co_off:co_off + c3, k:k + 1] = m[:, None]
        c1_off += c1
        co_off += c3


def _sa_msg(xyz_t, nxyz_t, nxyz_s, a, cs, branches, wts, s, sb):
    n = xyz_t.shape[2]
    raw = a is None
    co = sum(br[4] for br in branches)
    nsi = s // sb
    ncs4 = jnp.reshape(nxyz_s, (B, nsi, sb, 3))
    wt_specs = [pl.BlockSpec(w.shape, lambda b, si: (0, 0)) for w in wts]
    if raw:
        nct4 = jnp.transpose(
            jnp.reshape(nxyz_t, (B, 3, nsi, sb)), (0, 2, 1, 3))
        extra = [nct4]
        extra_specs = [pl.BlockSpec((1, 1, 3, sb), lambda b, si: (b, si, 0, 0))]
    else:
        ca = a.shape[1]
        cs4 = jnp.transpose(jnp.reshape(cs, (B, ca, nsi, sb)), (0, 2, 1, 3))
        extra = [a, cs4]
        extra_specs = [
            pl.BlockSpec((1, ca, n), lambda b, si: (b, 0, 0)),
            pl.BlockSpec((1, 1, ca, sb), lambda b, si: (b, si, 0, 0)),
        ]
    out4 = pl.pallas_call(
        functools.partial(_sa_body, branches=branches, n=n, sb=sb, raw=raw),
        grid=(B, nsi),
        in_specs=[
            pl.BlockSpec((1, 3, n), lambda b, si: (b, 0, 0)),
            pl.BlockSpec((1, 1, sb, 3), lambda b, si: (b, si, 0, 0)),
        ] + extra_specs + wt_specs,
        out_specs=pl.BlockSpec((1, 1, co, sb), lambda b, si: (b, si, 0, 0)),
        out_shape=jax.ShapeDtypeStruct((B, nsi, co, sb), jnp.float32),
    )(xyz_t, ncs4, *extra, *wts)
    return jnp.reshape(jnp.transpose(out4, (0, 2, 1, 3)), (B, co, s))


def _sa3_body(xyz_ref, pts_ref, w1_ref, w2_ref, w3_ref, bias_ref, out_ref):
    h = jnp.concatenate([xyz_ref[0], pts_ref[0]], axis=0)
    bias = bias_ref[...]
    b1 = bias[0:256, :]
    b2 = bias[256:256 + 512, :]
    b3 = bias[256 + 512:256 + 512 + 1024, :]
    h = _relu(_dot(w1_ref[...], h) + b1)
    h = _relu(_dot(w2_ref[...], h) + b2)
    h = _relu(_dot(w3_ref[...], h) + b3)
    out_ref[0] = jnp.max(h, axis=1)[:, None]


def _fc_body(p_ref, w1_ref, w2_ref, w3_ref, b1_ref, b2_ref, b3_ref, out_ref):
    f = p_ref[...]
    f = _relu(_dot(f, w1_ref[...]) + b1_ref[...])
    f = _relu(_dot(f, w2_ref[...]) + b2_ref[...])
    f = _relu(_dot(f, w3_ref[...]) + b3_ref[...])
    out_ref[...] = f


def _stack_wts(mlps, with_w1=False):
    out = []
    for mlp in mlps:
        (w1, b1), (w2, b2), (w3, b3) = mlp
        bias = jnp.concatenate([b1, b2, b3])[:, None]
        if with_w1:
            out += [w1.T, w2.T, w3.T, bias]
        else:
            out += [w2.T, w3.T, bias]
    return out


def kernel(inputs, sa1, sa2, sa3, fc):
    xyz0_t = jnp.transpose(inputs, (0, 2, 1))

    nxyz1_t, nxyz1_s = _fps(xyz0_t, 512)
    br1 = (
        (0.1 * 0.1, 16, 32, 32, 64),
        (0.2 * 0.2, 32, 64, 64, 128),
        (0.4 * 0.4, 128, 64, 96, 128),
    )
    pts1 = _sa_msg(xyz0_t, nxyz1_t, nxyz1_s, None, None, br1,
                   _stack_wts(sa1, with_w1=True), 512, 32)

    nxyz2_t, nxyz2_s = _fps(nxyz1_t, 128)
    wx2 = jnp.concatenate([mlp[0][0][:3, :].T for mlp in sa2], axis=0)
    wf2 = jnp.concatenate([mlp[0][0][3:, :].T for mlp in sa2], axis=0)
    a2, cs2 = pl.pallas_call(
        _proj2_body,
        grid=(B,),
        in_specs=[
            pl.BlockSpec((1, 3, 512), lambda b: (b, 0, 0)),
            pl.BlockSpec((1, 320, 512), lambda b: (b, 0, 0)),
            pl.BlockSpec((1, 3, 128), lambda b: (b, 0, 0)),
            pl.BlockSpec(wx2.shape, lambda b: (0, 0)),
            pl.BlockSpec(wf2.shape, lambda b: (0, 0)),
        ],
        out_specs=(
            pl.BlockSpec((1, 320, 512), lambda b: (b, 0, 0)),
            pl.BlockSpec((1, 320, 128), lambda b: (b, 0, 0)),
        ),
        out_shape=(
            jax.ShapeDtypeStruct((B, 320, 512), jnp.float32),
            jax.ShapeDtypeStruct((B, 320, 128), jnp.float32),
        ),
    )(nxyz1_t, pts1, nxyz2_t, wx2, wf2)
    br2 = (
        (0.2 * 0.2, 32, 64, 64, 128),
        (0.4 * 0.4, 64, 128, 128, 256),
        (0.8 * 0.8, 128, 128, 128, 256),
    )
    pts2 = _sa_msg(nxyz1_t, nxyz2_t, nxyz2_s, a2, cs2, br2,
                   _stack_wts(sa2), 128, 32)

    (w1, b1), (w2, b2), (w3, b3) = sa3
    bias3 = jnp.concatenate([b1, b2, b3])[:, None]
    pooled = pl.pallas_call(
        _sa3_body,
        grid=(B,),
        in_specs=[
            pl.BlockSpec((1, 3, 128), lambda b: (b, 0, 0)),
            pl.BlockSpec((1, 640, 128), lambda b: (b, 0, 0)),
            pl.BlockSpec((256, 643), lambda b: (0, 0)),
            pl.BlockSpec((512, 256), lambda b: (0, 0)),
            pl.BlockSpec((1024, 512), lambda b: (0, 0)),
            pl.BlockSpec((1792, 1), lambda b: (0, 0)),
        ],
        out_specs=pl.BlockSpec((1, 1024, 1), lambda b: (b, 0, 0)),
        out_shape=jax.ShapeDtypeStruct((B, 1024, 1), jnp.float32),
    )(nxyz2_t, pts2, w1.T, w2.T, w3.T, bias3)

    (fw1, fb1), (fw2, fb2), (fw3, fb3) = fc
    f = pl.pallas_call(
        _fc_body,
        out_shape=jax.ShapeDtypeStruct((B, 128), jnp.float32),
    )(pooled[:, :, 0], fw1, fw2, fw3, fb1[None, :], fb2[None, :], fb3[None, :])

    return (jnp.zeros((B, 1, 3), jnp.float32), f)


# speedup vs baseline: 12.6377x; 1.2263x over previous
import functools

import jax
import jax.numpy as jnp
from jax.experimental import pallas as pl

B = 8
N0 = 2048

def _relu(x):
    return jnp.maximum(x, 0.0)


def _dot(a, b):
    return jax.lax.dot_general(
        a, b, (((1,), (0,)), ((), ())), preferred_element_type=jnp.float32)


def _chunked_gather(tab, idx):
    n = tab.shape[1]
    lane = jnp.bitwise_and(idx, 127)
    cb = jnp.right_shift(idx, 7)
    parts = [
        jnp.take_along_axis(tab[:, c * 128:(c + 1) * 128], lane, axis=1)
        for c in range(n // 128)
    ]
    level = 0
    while len(parts) > 1:
        sel_hi = jnp.bitwise_and(jnp.right_shift(cb, level), 1) == 1
        parts = [
            jnp.where(sel_hi, parts[i + 1], parts[i])
            if i + 1 < len(parts) else parts[i]
            for i in range(0, len(parts), 2)
        ]
        level += 1
    return parts[0]


def _fps_body(xyz_ref, ns_ref, *, npoint, n):
    X = xyz_ref[:, 0, :]
    Y = xyz_ref[:, 1, :]
    Z = xyz_ref[:, 2, :]
    lanes = jax.lax.broadcasted_iota(jnp.int32, (B, n), 1)

    def body(j, carry):
        dist, far = carry
        rows = []
        for _ in range(8):
            oh = lanes == far
            cx = jnp.sum(jnp.where(oh, X, 0.0), axis=1, keepdims=True)
            cy = jnp.sum(jnp.where(oh, Y, 0.0), axis=1, keepdims=True)
            cz = jnp.sum(jnp.where(oh, Z, 0.0), axis=1, keepdims=True)
            rows.append(jnp.concatenate([cx, cy, cz], axis=1)[:, None, :])
            d = (X - cx) ** 2 + (Y - cy) ** 2 + (Z - cz) ** 2
            dist = jnp.minimum(dist, d)
            m = jnp.max(dist, axis=1, keepdims=True)
            far = jnp.min(jnp.where(dist == m, lanes, n), axis=1,
                          keepdims=True)
        blk = jnp.concatenate(rows, axis=1)
        ns_ref[:, pl.ds(pl.multiple_of(j * 8, 8), 8), :] = blk
        return dist, far

    dist0 = jnp.full((B, n), 1e10, dtype=jnp.float32)
    far0 = jnp.zeros((B, 1), dtype=jnp.int32)
    jax.lax.fori_loop(0, npoint // 8, body, (dist0, far0))


def _fps(xyz_t, npoint):
    n = xyz_t.shape[2]
    nxyz_s = pl.pallas_call(
        functools.partial(_fps_body, npoint=npoint, n=n),
        out_shape=jax.ShapeDtypeStruct((B, npoint, 3), jnp.float32),
    )(xyz_t)
    return jnp.transpose(nxyz_s, (0, 2, 1)), nxyz_s


def _proj1_body(xyz_ref, nxyz_ref, wx_ref, a_ref, cs_ref):
    wx = wx_ref[...]
    a_ref[0] = _dot(wx, xyz_ref[0])
    cs_ref[0] = _dot(wx, nxyz_ref[0])


def _proj2_body(xyz_ref, pts_ref, nxyz_ref, wx_ref, wf_ref, a_ref, cs_ref):
    wx = wx_ref[...]
    a_ref[0] = _dot(wx, xyz_ref[0]) + _dot(wf_ref[...], pts_ref[0])
    cs_ref[0] = _dot(wx, nxyz_ref[0])


def _sa_body(xyz_ref, ncs_ref, *refs, branches, n, sb, raw):
    nb = len(branches)
    npb = 4 if raw else 3
    if raw:
        nct_ref = refs[0]
        wts = refs[1:1 + npb * nb]
        out_ref = refs[1 + npb * nb]
    else:
        a_ref, cs_ref = refs[0], refs[1]
        wts = refs[2:2 + npb * nb]
        out_ref = refs[2 + npb * nb]

    xyz = xyz_ref[0]
    X = xyz[0:1, :]
    Y = xyz[1:2, :]
    Z = xyz[2:3, :]
    nc = ncs_ref[0, 0]
    cx = nc[:, 0:1]
    cy = nc[:, 1:2]
    cz = nc[:, 2:3]
    sqr = (cx - X) ** 2 + (cy - Y) ** 2 + (cz - Z) ** 2

    iota0 = jax.lax.broadcasted_iota(jnp.int32, (128, 128), 0)
    iota1 = jax.lax.broadcasted_iota(jnp.int32, (128, 128), 1)
    ut = (iota0 <= iota1).astype(jnp.float32)
    nch = n // 128
    iota0c = jax.lax.broadcasted_iota(jnp.int32, (nch, nch), 0)
    iota1c = jax.lax.broadcasted_iota(jnp.int32, (nch, nch), 1)
    utx = (iota0c < iota1c).astype(jnp.float32)

    if raw:
        nct = nct_ref[0, 0]
        a = xyz
    else:
        a = a_ref[0]
        cs = cs_ref[0, 0]

    c1_off = 0
    co_off = 0
    nsteps = 7
    for bi in range(nb):
        r2, ns, c1, c2, c3 = branches[bi]
        if raw:
            w1t_ref, w2t_ref, w3t_ref, bias_ref = wts[npb * bi:npb * bi + npb]
        else:
            w2t_ref, w3t_ref, bias_ref = wts[npb * bi:npb * bi + npb]
        mask = sqr <= jnp.float32(r2)
        mf = mask.astype(jnp.float32)
        pcs = [_dot(mf[:, c * 128:(c + 1) * 128], ut) for c in range(nch)]
        sums = jnp.concatenate([p[:, -1:] for p in pcs], axis=1)
        offs = _dot(sums, utx)
        rank = jnp.concatenate(
            [pcs[c] + offs[:, c:c + 1] for c in range(nch)], axis=1)
        count = offs[:, -1:] + sums[:, -1:]

        tgt = (jax.lax.broadcasted_iota(jnp.int32, (sb, ns), 1) + 1
               ).astype(jnp.float32)
        ccum = offs + sums
        cid = jnp.zeros((sb, ns), jnp.int32)
        for c in range(nch):
            cid = cid + (ccum[:, c:c + 1] < tgt).astype(jnp.int32)
        lo = cid * 128
        hi = jnp.minimum(lo + 127, n)
        for _ in range(nsteps):
            act = lo < hi
            mid = (lo + hi) // 2
            rmid = _chunked_gather(rank, jnp.minimum(mid, n - 1))
            pred = act & (rmid < tgt)
            lo = jnp.where(pred, mid + 1, lo)
            hi = jnp.where(act & (~pred), mid, hi)
        valid = tgt <= count
        gidx = jnp.where(valid, lo, jnp.broadcast_to(lo[:, 0:1], (sb, ns)))
        gidx = jnp.minimum(gidx, n - 1)

        bias = bias_ref[...]
        b1 = bias[0:c1, :]
        b2 = bias[c1:c1 + c2, :]
        b3 = bias[c1 + c2:c1 + c2 + c3, :]
        if raw:
            cols = []
            for k in range(sb):
                idxk = jnp.broadcast_to(gidx[k:k + 1, :], (3, ns))
                g = _chunked_gather(xyz, idxk)
                ck = nct[:, k:k + 1]
                cols.append(g - jnp.broadcast_to(ck, (3, ns)))
            pre = jnp.concatenate(cols, axis=1)
            h1 = _relu(_dot(w1t_ref[...], pre) + b1)
        else:
            ab = a[c1_off:c1_off + c1, :]
            cols = []
            for k in range(sb):
                idxk = jnp.broadcast_to(gidx[k:k + 1, :], (c1, ns))
                g = _chunked_gather(ab, idxk)
                ck = cs[c1_off:c1_off + c1, k:k + 1]
                cols.append(g - jnp.broadcast_to(ck, (c1, ns)))
            h1 = _relu(jnp.concatenate(cols, axis=1) + b1)
        h2 = _relu(_dot(w2t_ref[...], h1) + b2)
        h3 = _relu(_dot(w3t_ref[...], h2) + b3)
        for k in range(sb):
            m = jnp.max(h3[:, k * ns:(k + 1) * ns], axis=1)
            out_ref[0, 0, co_off:co_off + c3, k:k + 1] = m[:, None]
        c1_off += c1
        co_off += c3


def _sa_msg(xyz_t, nxyz_t, nxyz_s, a, cs, branches, wts, s, sb):
    n = xyz_t.shape[2]
    raw = a is None
    co = sum(br[4] for br in branches)
    nsi = s // sb
    ncs4 = jnp.reshape(nxyz_s, (B, nsi, sb, 3))
    wt_specs = [pl.BlockSpec(w.shape, lambda b, si: (0, 0)) for w in wts]
    if raw:
        nct4 = jnp.transpose(
            jnp.reshape(nxyz_t, (B, 3, nsi, sb)), (0, 2, 1, 3))
        extra = [nct4]
        extra_specs = [pl.BlockSpec((1, 1, 3, sb), lambda b, si: (b, si, 0, 0))]
    else:
        ca = a.shape[1]
        cs4 = jnp.transpose(jnp.reshape(cs, (B, ca, nsi, sb)), (0, 2, 1, 3))
        extra = [a, cs4]
        extra_specs = [
            pl.BlockSpec((1, ca, n), lambda b, si: (b, 0, 0)),
            pl.BlockSpec((1, 1, ca, sb), lambda b, si: (b, si, 0, 0)),
        ]
    out4 = pl.pallas_call(
        functools.partial(_sa_body, branches=branches, n=n, sb=sb, raw=raw),
        grid=(B, nsi),
        in_specs=[
            pl.BlockSpec((1, 3, n), lambda b, si: (b, 0, 0)),
            pl.BlockSpec((1, 1, sb, 3), lambda b, si: (b, si, 0, 0)),
        ] + extra_specs + wt_specs,
        out_specs=pl.BlockSpec((1, 1, co, sb), lambda b, si: (b, si, 0, 0)),
        out_shape=jax.ShapeDtypeStruct((B, nsi, co, sb), jnp.float32),
    )(xyz_t, ncs4, *extra, *wts)
    return jnp.reshape(jnp.transpose(out4, (0, 2, 1, 3)), (B, co, s))


def _sa3_body(xyz_ref, pts_ref, w1_ref, w2_ref, w3_ref, bias_ref, out_ref):
    h = jnp.concatenate([xyz_ref[0], pts_ref[0]], axis=0)
    bias = bias_ref[...]
    b1 = bias[0:256, :]
    b2 = bias[256:256 + 512, :]
    b3 = bias[256 + 512:256 + 512 + 1024, :]
    h = _relu(_dot(w1_ref[...], h) + b1)
    h = _relu(_dot(w2_ref[...], h) + b2)
    h = _relu(_dot(w3_ref[...], h) + b3)
    out_ref[0] = jnp.max(h, axis=1)[:, None]


def _fc_body(p_ref, w1_ref, w2_ref, w3_ref, b1_ref, b2_ref, b3_ref, out_ref):
    f = p_ref[...]
    f = _relu(_dot(f, w1_ref[...]) + b1_ref[...])
    f = _relu(_dot(f, w2_ref[...]) + b2_ref[...])
    f = _relu(_dot(f, w3_ref[...]) + b3_ref[...])
    out_ref[...] = f


def _stack_wts(mlps, with_w1=False):
    out = []
    for mlp in mlps:
        (w1, b1), (w2, b2), (w3, b3) = mlp
        bias = jnp.concatenate([b1, b2, b3])[:, None]
        if with_w1:
            out += [w1.T, w2.T, w3.T, bias]
        else:
            out += [w2.T, w3.T, bias]
    return out


def kernel(inputs, sa1, sa2, sa3, fc):
    xyz0_t = jnp.transpose(inputs, (0, 2, 1))

    nxyz1_t, nxyz1_s = _fps(xyz0_t, 512)
    br1 = (
        (0.1 * 0.1, 16, 32, 32, 64),
        (0.2 * 0.2, 32, 64, 64, 128),
        (0.4 * 0.4, 128, 64, 96, 128),
    )
    pts1 = _sa_msg(xyz0_t, nxyz1_t, nxyz1_s, None, None, br1,
                   _stack_wts(sa1, with_w1=True), 512, 32)

    nxyz2_t, nxyz2_s = _fps(nxyz1_t, 128)
    wx2 = jnp.concatenate([mlp[0][0][:3, :].T for mlp in sa2], axis=0)
    wf2 = jnp.concatenate([mlp[0][0][3:, :].T for mlp in sa2], axis=0)
    a2, cs2 = pl.pallas_call(
        _proj2_body,
        grid=(B,),
        in_specs=[
            pl.BlockSpec((1, 3, 512), lambda b: (b, 0, 0)),
            pl.BlockSpec((1, 320, 512), lambda b: (b, 0, 0)),
            pl.BlockSpec((1, 3, 128), lambda b: (b, 0, 0)),
            pl.BlockSpec(wx2.shape, lambda b: (0, 0)),
            pl.BlockSpec(wf2.shape, lambda b: (0, 0)),
        ],
        out_specs=(
            pl.BlockSpec((1, 320, 512), lambda b: (b, 0, 0)),
            pl.BlockSpec((1, 320, 128), lambda b: (b, 0, 0)),
        ),
        out_shape=(
            jax.ShapeDtypeStruct((B, 320, 512), jnp.float32),
            jax.ShapeDtypeStruct((B, 320, 128), jnp.float32),
        ),
    )(nxyz1_t, pts1, nxyz2_t, wx2, wf2)
    br2 = (
        (0.2 * 0.2, 32, 64, 64, 128),
        (0.4 * 0.4, 64, 128, 128, 256),
        (0.8 * 0.8, 128, 128, 128, 256),
    )
    pts2 = _sa_msg(nxyz1_t, nxyz2_t, nxyz2_s, a2, cs2, br2,
                   _stack_wts(sa2), 128, 32)

    (w1, b1), (w2, b2), (w3, b3) = sa3
    bias3 = jnp.concatenate([b1, b2, b3])[:, None]
    pooled = pl.pallas_call(
        _sa3_body,
        grid=(B,),
        in_specs=[
            pl.BlockSpec((1, 3, 128), lambda b: (b, 0, 0)),
            pl.BlockSpec((1, 640, 128), lambda b: (b, 0, 0)),
            pl.BlockSpec((256, 643), lambda b: (0, 0)),
            pl.BlockSpec((512, 256), lambda b: (0, 0)),
            pl.BlockSpec((1024, 512), lambda b: (0, 0)),
            pl.BlockSpec((1792, 1), lambda b: (0, 0)),
        ],
        out_specs=pl.BlockSpec((1, 1024, 1), lambda b: (b, 0, 0)),
        out_shape=jax.ShapeDtypeStruct((B, 1024, 1), jnp.float32),
    )(nxyz2_t, pts2, w1.T, w2.T, w3.T, bias3)

    (fw1, fb1), (fw2, fb2), (fw3, fb3) = fc
    f = pl.pallas_call(
        _fc_body,
        out_shape=jax.ShapeDtypeStruct((B, 128), jnp.float32),
    )(pooled[:, :, 0], fw1, fw2, fw3, fb1[None, :], fb2[None, :], fb3[None, :])

    return (jnp.zeros((B, 1, 3), jnp.float32), f)


# speedup vs baseline: 13.0056x; 1.0291x over previous
import functools

import jax
import jax.numpy as jnp
from jax.experimental import pallas as pl

B = 8
N0 = 2048

def _relu(x):
    return jnp.maximum(x, 0.0)


def _dot(a, b):
    return jax.lax.dot_general(
        a, b, (((1,), (0,)), ((), ())), preferred_element_type=jnp.float32)


def _chunked_gather(tab, idx):
    n = tab.shape[1]
    lane = jnp.bitwise_and(idx, 127)
    cb = jnp.right_shift(idx, 7)
    parts = [
        jnp.take_along_axis(tab[:, c * 128:(c + 1) * 128], lane, axis=1)
        for c in range(n // 128)
    ]
    level = 0
    while len(parts) > 1:
        sel_hi = jnp.bitwise_and(jnp.right_shift(cb, level), 1) == 1
        parts = [
            jnp.where(sel_hi, parts[i + 1], parts[i])
            if i + 1 < len(parts) else parts[i]
            for i in range(0, len(parts), 2)
        ]
        level += 1
    return parts[0]


def _fps_body(xyz_ref, ns_ref, *, npoint, n):
    X = xyz_ref[:, 0, :]
    Y = xyz_ref[:, 1, :]
    Z = xyz_ref[:, 2, :]
    lanes = jax.lax.broadcasted_iota(jnp.int32, (B, n), 1)

    def body(j, carry):
        dist, far = carry
        rows = []
        for _ in range(8):
            oh = lanes == far
            cx = jnp.sum(jnp.where(oh, X, 0.0), axis=1, keepdims=True)
            cy = jnp.sum(jnp.where(oh, Y, 0.0), axis=1, keepdims=True)
            cz = jnp.sum(jnp.where(oh, Z, 0.0), axis=1, keepdims=True)
            rows.append(jnp.concatenate([cx, cy, cz], axis=1)[:, None, :])
            d = (X - cx) ** 2 + (Y - cy) ** 2 + (Z - cz) ** 2
            dist = jnp.minimum(dist, d)
            m = jnp.max(dist, axis=1, keepdims=True)
            far = jnp.min(jnp.where(dist == m, lanes, n), axis=1,
                          keepdims=True)
        blk = jnp.concatenate(rows, axis=1)
        ns_ref[:, pl.ds(pl.multiple_of(j * 8, 8), 8), :] = blk
        return dist, far

    dist0 = jnp.full((B, n), 1e10, dtype=jnp.float32)
    far0 = jnp.zeros((B, 1), dtype=jnp.int32)
    jax.lax.fori_loop(0, npoint // 8, body, (dist0, far0))


def _fps(xyz_t, npoint):
    n = xyz_t.shape[2]
    nxyz_s = pl.pallas_call(
        functools.partial(_fps_body, npoint=npoint, n=n),
        out_shape=jax.ShapeDtypeStruct((B, npoint, 3), jnp.float32),
    )(xyz_t)
    return jnp.transpose(nxyz_s, (0, 2, 1)), nxyz_s


def _proj1_body(xyz_ref, nxyz_ref, wx_ref, a_ref, cs_ref):
    wx = wx_ref[...]
    a_ref[0] = _dot(wx, xyz_ref[0])
    cs_ref[0] = _dot(wx, nxyz_ref[0])


def _proj2_body(xyz_ref, pts_ref, nxyz_ref, wx_ref, wf_ref, a_ref, cs_ref):
    wx = wx_ref[...]
    a_ref[0] = _dot(wx, xyz_ref[0]) + _dot(wf_ref[...], pts_ref[0])
    cs_ref[0] = _dot(wx, nxyz_ref[0])


def _sa_body(xyz_ref, ncs_ref, *refs, branches, n, sb, raw):
    nb = len(branches)
    npb = 4 if raw else 3
    if raw:
        nct_ref = refs[0]
        wts = refs[1:1 + npb * nb]
        out_ref = refs[1 + npb * nb]
    else:
        a_ref, cs_ref = refs[0], refs[1]
        wts = refs[2:2 + npb * nb]
        out_ref = refs[2 + npb * nb]

    xyz = xyz_ref[0]
    X = xyz[0:1, :]
    Y = xyz[1:2, :]
    Z = xyz[2:3, :]
    nc = ncs_ref[0, 0]
    cx = nc[:, 0:1]
    cy = nc[:, 1:2]
    cz = nc[:, 2:3]
    sqr = (cx - X) ** 2 + (cy - Y) ** 2 + (cz - Z) ** 2

    iota0 = jax.lax.broadcasted_iota(jnp.int32, (128, 128), 0)
    iota1 = jax.lax.broadcasted_iota(jnp.int32, (128, 128), 1)
    ut = (iota0 <= iota1).astype(jnp.float32)
    nch = n // 128
    iota0c = jax.lax.broadcasted_iota(jnp.int32, (nch, nch), 0)
    iota1c = jax.lax.broadcasted_iota(jnp.int32, (nch, nch), 1)
    utx = (iota0c < iota1c).astype(jnp.float32)

    if raw:
        nct = nct_ref[0, 0]
        a = xyz
    else:
        a = a_ref[0]
        cs = cs_ref[0, 0]

    c1_off = 0
    co_off = 0
    nsteps = 7
    for bi in range(nb):
        r2, ns, c1, c2, c3 = branches[bi]
        if raw:
            w1t_ref, w2t_ref, w3t_ref, bias_ref = wts[npb * bi:npb * bi + npb]
        else:
            w2t_ref, w3t_ref, bias_ref = wts[npb * bi:npb * bi + npb]
        mask = sqr <= jnp.float32(r2)
        mf = mask.astype(jnp.float32)
        pcs = [_dot(mf[:, c * 128:(c + 1) * 128], ut) for c in range(nch)]
        sums = jnp.concatenate([p[:, -1:] for p in pcs], axis=1)
        offs = _dot(sums, utx)
        rank = jnp.concatenate(
            [pcs[c] + offs[:, c:c + 1] for c in range(nch)], axis=1)
        count = offs[:, -1:] + sums[:, -1:]

        tgt = (jax.lax.broadcasted_iota(jnp.int32, (sb, ns), 1) + 1
               ).astype(jnp.float32)
        ccum = offs + sums
        cid = jnp.zeros((sb, ns), jnp.int32)
        for c in range(nch):
            cid = cid + (ccum[:, c:c + 1] < tgt).astype(jnp.int32)
        lo = cid * 128
        hi = jnp.minimum(lo + 127, n)
        for _ in range(nsteps):
            act = lo < hi
            mid = (lo + hi) // 2
            rmid = _chunked_gather(rank, jnp.minimum(mid, n - 1))
            pred = act & (rmid < tgt)
            lo = jnp.where(pred, mid + 1, lo)
            hi = jnp.where(act & (~pred), mid, hi)
        valid = tgt <= count
        gidx = jnp.where(valid, lo, jnp.broadcast_to(lo[:, 0:1], (sb, ns)))
        gidx = jnp.minimum(gidx, n - 1)

        bias = bias_ref[...]
        b1 = bias[0:c1, :]
        b2 = bias[c1:c1 + c2, :]
        b3 = bias[c1 + c2:c1 + c2 + c3, :]
        if raw:
            cols = []
            for k in range(sb):
                idxk = jnp.broadcast_to(gidx[k:k + 1, :], (3, ns))
                g = _chunked_gather(xyz, idxk)
                ck = nct[:, k:k + 1]
                cols.append(g - jnp.broadcast_to(ck, (3, ns)))
            pre = jnp.concatenate(cols, axis=1)
            h1 = _relu(_dot(w1t_ref[...], pre) + b1)
        else:
            ab = a[c1_off:c1_off + c1, :]
            cols = []
            for k in range(sb):
                idxk = jnp.broadcast_to(gidx[k:k + 1, :], (c1, ns))
                g = _chunked_gather(ab, idxk)
                ck = cs[c1_off:c1_off + c1, k:k + 1]
                cols.append(g - jnp.broadcast_to(ck, (c1, ns)))
            h1 = _relu(jnp.concatenate(cols, axis=1) + b1)
        h2 = _relu(_dot(w2t_ref[...], h1) + b2)
        h3 = _relu(_dot(w3t_ref[...], h2) + b3)
        for k in range(sb):
            m = jnp.max(h3[:, k * ns:(k + 1) * ns], axis=1)
            out_ref[0, 0, co_off:co_off + c3, k:k + 1] = m[:, None]
        c1_off += c1
        co_off += c3


def _sa_msg(xyz_t, nxyz_t, nxyz_s, a, cs, branches, wts, s, sb):
    n = xyz_t.shape[2]
    raw = a is None
    co = sum(br[4] for br in branches)
    nsi = s // sb
    ncs4 = jnp.reshape(nxyz_s, (B, nsi, sb, 3))
    wt_specs = [pl.BlockSpec(w.shape, lambda b, si: (0, 0)) for w in wts]
    if raw:
        nct4 = jnp.transpose(
            jnp.reshape(nxyz_t, (B, 3, nsi, sb)), (0, 2, 1, 3))
        extra = [nct4]
        extra_specs = [pl.BlockSpec((1, 1, 3, sb), lambda b, si: (b, si, 0, 0))]
    else:
        ca = a.shape[1]
        cs4 = jnp.transpose(jnp.reshape(cs, (B, ca, nsi, sb)), (0, 2, 1, 3))
        extra = [a, cs4]
        extra_specs = [
            pl.BlockSpec((1, ca, n), lambda b, si: (b, 0, 0)),
            pl.BlockSpec((1, 1, ca, sb), lambda b, si: (b, si, 0, 0)),
        ]
    out4 = pl.pallas_call(
        functools.partial(_sa_body, branches=branches, n=n, sb=sb, raw=raw),
        grid=(B, nsi),
        in_specs=[
            pl.BlockSpec((1, 3, n), lambda b, si: (b, 0, 0)),
            pl.BlockSpec((1, 1, sb, 3), lambda b, si: (b, si, 0, 0)),
        ] + extra_specs + wt_specs,
        out_specs=pl.BlockSpec((1, 1, co, sb), lambda b, si: (b, si, 0, 0)),
        out_shape=jax.ShapeDtypeStruct((B, nsi, co, sb), jnp.float32),
    )(xyz_t, ncs4, *extra, *wts)
    return jnp.reshape(jnp.transpose(out4, (0, 2, 1, 3)), (B, co, s))


def _sa3_body(xyz_ref, pts_ref, w1_ref, w2_ref, w3_ref, bias_ref, out_ref):
    h = jnp.concatenate([xyz_ref[0], pts_ref[0]], axis=0)
    bias = bias_ref[...]
    b1 = bias[0:256, :]
    b2 = bias[256:256 + 512, :]
    b3 = bias[256 + 512:256 + 512 + 1024, :]
    h = _relu(_dot(w1_ref[...], h) + b1)
    h = _relu(_dot(w2_ref[...], h) + b2)
    h = _relu(_dot(w3_ref[...], h) + b3)
    out_ref[0] = jnp.max(h, axis=1)[:, None]


def _fc_body(p_ref, w1_ref, w2_ref, w3_ref, b1_ref, b2_ref, b3_ref, out_ref):
    f = p_ref[...]
    f = _relu(_dot(f, w1_ref[...]) + b1_ref[...])
    f = _relu(_dot(f, w2_ref[...]) + b2_ref[...])
    f = _relu(_dot(f, w3_ref[...]) + b3_ref[...])
    out_ref[...] = f


def _stack_wts(mlps, with_w1=False):
    out = []
    for mlp in mlps:
        (w1, b1), (w2, b2), (w3, b3) = mlp
        bias = jnp.concatenate([b1, b2, b3])[:, None]
        if with_w1:
            out += [w1.T, w2.T, w3.T, bias]
        else:
            out += [w2.T, w3.T, bias]
    return out


def kernel(inputs, sa1, sa2, sa3, fc):
    xyz0_t = jnp.transpose(inputs, (0, 2, 1))

    nxyz1_t, nxyz1_s = _fps(xyz0_t, 512)
    br1 = (
        (0.1 * 0.1, 16, 32, 32, 64),
        (0.2 * 0.2, 32, 64, 64, 128),
        (0.4 * 0.4, 128, 64, 96, 128),
    )
    pts1 = _sa_msg(xyz0_t, nxyz1_t, nxyz1_s, None, None, br1,
                   _stack_wts(sa1, with_w1=True), 512, 64)

    nxyz2_t, nxyz2_s = _fps(nxyz1_t, 128)
    wx2 = jnp.concatenate([mlp[0][0][:3, :].T for mlp in sa2], axis=0)
    wf2 = jnp.concatenate([mlp[0][0][3:, :].T for mlp in sa2], axis=0)
    a2, cs2 = pl.pallas_call(
        _proj2_body,
        grid=(B,),
        in_specs=[
            pl.BlockSpec((1, 3, 512), lambda b: (b, 0, 0)),
            pl.BlockSpec((1, 320, 512), lambda b: (b, 0, 0)),
            pl.BlockSpec((1, 3, 128), lambda b: (b, 0, 0)),
            pl.BlockSpec(wx2.shape, lambda b: (0, 0)),
            pl.BlockSpec(wf2.shape, lambda b: (0, 0)),
        ],
        out_specs=(
            pl.BlockSpec((1, 320, 512), lambda b: (b, 0, 0)),
            pl.BlockSpec((1, 320, 128), lambda b: (b, 0, 0)),
        ),
        out_shape=(
            jax.ShapeDtypeStruct((B, 320, 512), jnp.float32),
            jax.ShapeDtypeStruct((B, 320, 128), jnp.float32),
        ),
    )(nxyz1_t, pts1, nxyz2_t, wx2, wf2)
    br2 = (
        (0.2 * 0.2, 32, 64, 64, 128),
        (0.4 * 0.4, 64, 128, 128, 256),
        (0.8 * 0.8, 128, 128, 128, 256),
    )
    pts2 = _sa_msg(nxyz1_t, nxyz2_t, nxyz2_s, a2, cs2, br2,
                   _stack_wts(sa2), 128, 32)

    (w1, b1), (w2, b2), (w3, b3) = sa3
    bias3 = jnp.concatenate([b1, b2, b3])[:, None]
    pooled = pl.pallas_call(
        _sa3_body,
        grid=(B,),
        in_specs=[
            pl.BlockSpec((1, 3, 128), lambda b: (b, 0, 0)),
            pl.BlockSpec((1, 640, 128), lambda b: (b, 0, 0)),
            pl.BlockSpec((256, 643), lambda b: (0, 0)),
            pl.BlockSpec((512, 256), lambda b: (0, 0)),
            pl.BlockSpec((1024, 512), lambda b: (0, 0)),
            pl.BlockSpec((1792, 1), lambda b: (0, 0)),
        ],
        out_specs=pl.BlockSpec((1, 1024, 1), lambda b: (b, 0, 0)),
        out_shape=jax.ShapeDtypeStruct((B, 1024, 1), jnp.float32),
    )(nxyz2_t, pts2, w1.T, w2.T, w3.T, bias3)

    (fw1, fb1), (fw2, fb2), (fw3, fb3) = fc
    f = pl.pallas_call(
        _fc_body,
        out_shape=jax.ShapeDtypeStruct((B, 128), jnp.float32),
    )(pooled[:, :, 0], fw1, fw2, fw3, fb1[None, :], fb2[None, :], fb3[None, :])

    return (jnp.zeros((B, 1, 3), jnp.float32), f)
